# baseline (speedup 1.0000x reference)
import jax
import jax.numpy as jnp
from jax import lax
from jax.experimental import pallas as pl
from jax.experimental.pallas import tpu as pltpu
from jax.experimental.pallas import tpu_sc as plsc

B, C, N = 4, 128, 128
PAD = 128
PW = N + 2 * PAD
NRES = 8
NCEN = N // NRES
TW = 192
NT = 8
CPB = N // NT


def _sc_body(feat_hbm, o256_hbm, raw, buf3, sem):
    cid = lax.axis_index("c")
    sid = lax.axis_index("s")
    wid = sid * 2 + cid
    b = wid // NRES
    s = wid % NRES

    in_cp = pltpu.make_async_copy(feat_hbm.at[b], raw, sem)
    in_cp.start()
    lanes = lax.iota(jnp.int32, 16)
    zeros = jnp.zeros((16,), jnp.float32)

    def zero_row(c, carry):
        ct = c // 8
        ci = c % 8
        for k in (*range(7), *range(16, 24)):
            buf3[ct, ci, pl.ds(k * 16, 16)] = zeros
        return carry

    lax.fori_loop(0, C, zero_row, 0)
    in_cp.wait()

    def o256_copies(m):
        n = m * NRES + s
        c256 = pl.multiple_of(m * NRES, NRES)
        return (
            (buf3.at[:, :, pl.ds(c256, 128)], o256_hbm.at[b, n, :, 0]),
            (buf3.at[:, :, pl.ds(c256 + 128, 128)], o256_hbm.at[b, n, :, 1]),
        )

    def gather_row(c, carry):
        ct = c // 8
        ci = c % 8
        row_idx = jnp.full((16,), c, jnp.int32)
        for k in (7, 15):
            col = lanes + (k * 16 + s - PAD)
            ok = (col >= 0) & (col < N)
            v = plsc.load_gather(raw, [row_idx, jnp.clip(col, 0, N - 1)])
            buf3[ct, ci, pl.ds(k * 16, 16)] = jnp.where(ok, v, 0.0)
        for k in range(8, 15):
            col = lanes + (k * 16 + s - PAD)
            buf3[ct, ci, pl.ds(k * 16, 16)] = plsc.load_gather(raw, [row_idx, col])
        return carry

    lax.fori_loop(0, C, gather_row, 0)

    def fire_o256(m, carry):
        for src, dst in o256_copies(m):
            pltpu.async_copy(src, dst, sem)
        return carry

    def drain(m, carry):
        for src, dst in o256_copies(m):
            pltpu.make_async_copy(src, dst, sem).wait()
        return carry

    lax.fori_loop(0, NCEN, fire_o256, 0)
    lax.fori_loop(0, NCEN, drain, 0)


def _tc_body(feat_ref, o128a_ref, o128b_ref, o64_ref, padded):
    nt = pl.program_id(1)

    @pl.when(nt == 0)
    def _():
        padded[...] = jnp.zeros((C, PW), jnp.float32)
        padded[:, PAD:PAD + N] = feat_ref[0]

    n0 = nt * CPB
    ji = lax.broadcasted_iota(jnp.int32, (C, 128), 1)
    for i in range(CPB):
        col0 = n0 + i + PAD - 64
        k0 = pl.multiple_of((col0 // 128) * 128, 128)
        r = col0 % 128
        lo = padded[:, pl.ds(k0, 128)]
        hi = padded[:, pl.ds(k0 + 128, 128)]
        win = jnp.where(ji < 128 - r,
                        pltpu.roll(lo, -r, 1), pltpu.roll(hi, -r, 1))
        blk = win.reshape(C // 8, 8, 128)
        o128a_ref[0, i] = blk
        o128b_ref[0, i] = blk
        o64_ref[0, i] = jnp.swapaxes(win[:, 32:96], 0, 1)


@jax.jit
def _run(feature_seq):
    sc_out_type = jax.ShapeDtypeStruct((B, N, 16, 2, 8, 128), jnp.float32)
    mesh = plsc.VectorSubcoreMesh(core_axis_name="c", subcore_axis_name="s")
    sc_f = pl.kernel(
        _sc_body,
        out_type=sc_out_type,
        mesh=mesh,
        scratch_types=[
            pltpu.VMEM((C, N), jnp.float32),
            pltpu.VMEM((C // 8, 8, PW), jnp.float32),
            pltpu.SemaphoreType.DMA,
        ],
        compiler_params=pltpu.CompilerParams(
            use_tc_tiling_on_sc=False, needs_layout_passes=False),
    )
    o256t = sc_f(feature_seq)

    o128_sds = jax.ShapeDtypeStruct((B, N, 16, 8, 128), jnp.float32)
    o64_sds = jax.ShapeDtypeStruct((B, N, 64, 128), jnp.float32)
    o128a, o128b, o64t = pl.pallas_call(
        _tc_body,
        grid=(B, NT),
        in_specs=[pl.BlockSpec((1, C, N), lambda b, nt: (b, 0, 0))],
        out_specs=(
            pl.BlockSpec((1, CPB, 16, 8, 128), lambda b, nt: (b, nt, 0, 0, 0)),
            pl.BlockSpec((1, CPB, 16, 8, 128), lambda b, nt: (b, nt, 0, 0, 0)),
            pl.BlockSpec((1, CPB, 64, 128), lambda b, nt: (b, nt, 0, 0)),
        ),
        out_shape=(o128_sds, o128_sds, o64_sds),
        scratch_shapes=[pltpu.VMEM((C, PW), jnp.float32)],
    )(feature_seq)
    return o64t, o256t, o128a, o128b


def kernel(feature_seq):
    o64t, o256t, o128a, o128b = _run(feature_seq)
    o64 = jnp.transpose(o64t, (0, 1, 3, 2))
    o256 = jnp.transpose(o256t, (0, 1, 2, 4, 3, 5)).reshape(B, N, C, 256)
    return (o64, o256,
            o128a.reshape(B, N, C, 128), o128b.reshape(B, N, C, 128))


# speedup vs baseline: 1.3431x; 1.3431x over previous
import jax
import jax.numpy as jnp
from jax import lax
from jax.experimental import pallas as pl
from jax.experimental.pallas import tpu as pltpu
from jax.experimental.pallas import tpu_sc as plsc

B, C, N = 4, 128, 128
PAD = 128
PW = N + 2 * PAD
NRES = 8
NCEN = N // NRES
TW = 192
NT = 8
CPB = N // NT


def _sc_body(feat_hbm, o64_hbm, o256_hbm, raw, buf3, bufT, sem):
    cid = lax.axis_index("c")
    sid = lax.axis_index("s")
    wid = sid * 2 + cid
    b = wid // NRES
    s = wid % NRES

    in_cp = pltpu.make_async_copy(feat_hbm.at[b], raw, sem)
    in_cp.start()
    lanes = lax.iota(jnp.int32, 16)
    zeros = jnp.zeros((16,), jnp.float32)

    def zero_row(c, carry):
        ct = c // 8
        ci = c % 8
        for k in (*range(7), *range(16, 24)):
            buf3[ct, ci, pl.ds(k * 16, 16)] = zeros
        return carry

    def t_zero_row(r, carry):
        for cc in range(C // 16):
            bufT[r, pl.ds(cc * 16, 16)] = zeros
        return carry

    lax.fori_loop(0, C, zero_row, 0)
    lax.fori_loop(0, 32, t_zero_row, 0)
    lax.fori_loop(160, TW, t_zero_row, 0)
    in_cp.wait()

    def t_data_row(r, carry):
        col_idx = jnp.full((16,), r - 32, jnp.int32)
        for cc in range(C // 16):
            bufT[r, pl.ds(cc * 16, 16)] = plsc.load_gather(
                raw, [cc * 16 + lanes, col_idx])
        return carry

    lax.fori_loop(32, 160, t_data_row, 0)

    def o64_copy(m):
        n = m * NRES + s
        return (bufT.at[pl.ds(n, 64), :], o64_hbm.at[b, n])

    def o256_copies(m):
        n = m * NRES + s
        c256 = pl.multiple_of(m * NRES, NRES)
        return (
            (buf3.at[:, :, pl.ds(c256, 128)], o256_hbm.at[b, n, :, 0]),
            (buf3.at[:, :, pl.ds(c256 + 128, 128)], o256_hbm.at[b, n, :, 1]),
        )

    def fire_o64(m, carry):
        src, dst = o64_copy(m)
        pltpu.async_copy(src, dst, sem)
        return carry

    lax.fori_loop(0, NCEN, fire_o64, 0)

    def gather_row(c, carry):
        ct = c // 8
        ci = c % 8
        row_idx = jnp.full((16,), c, jnp.int32)
        for k in (7, 15):
            col = lanes + (k * 16 + s - PAD)
            ok = (col >= 0) & (col < N)
            v = plsc.load_gather(raw, [row_idx, jnp.clip(col, 0, N - 1)])
            buf3[ct, ci, pl.ds(k * 16, 16)] = jnp.where(ok, v, 0.0)
        for k in range(8, 15):
            col = lanes + (k * 16 + s - PAD)
            buf3[ct, ci, pl.ds(k * 16, 16)] = plsc.load_gather(raw, [row_idx, col])
        return carry

    lax.fori_loop(0, C, gather_row, 0)

    def fire_o256(m, carry):
        for src, dst in o256_copies(m):
            pltpu.async_copy(src, dst, sem)
        return carry

    def drain(m, carry):
        src, dst = o64_copy(m)
        pltpu.make_async_copy(src, dst, sem).wait()
        for src, dst in o256_copies(m):
            pltpu.make_async_copy(src, dst, sem).wait()
        return carry

    lax.fori_loop(0, NCEN, fire_o256, 0)
    lax.fori_loop(0, NCEN, drain, 0)


def _tc_body(feat_ref, o128a_ref, o128b_ref, padded):
    nt = pl.program_id(1)

    @pl.when(nt == 0)
    def _():
        padded[...] = jnp.zeros((C, PW), jnp.float32)
        padded[:, PAD:PAD + N] = feat_ref[0]

    n0 = nt * CPB
    ji = lax.broadcasted_iota(jnp.int32, (C, 128), 1)
    for i in range(CPB):
        col0 = n0 + i + PAD - 64
        k0 = pl.multiple_of((col0 // 128) * 128, 128)
        r = col0 % 128
        lo = padded[:, pl.ds(k0, 128)]
        hi = padded[:, pl.ds(k0 + 128, 128)]
        win = jnp.where(ji < 128 - r,
                        pltpu.roll(lo, -r, 1), pltpu.roll(hi, -r, 1))
        blk = win.reshape(C // 8, 8, 128)
        o128a_ref[0, i] = blk
        o128b_ref[0, i] = blk


@jax.jit
def _run(feature_seq):
    sc_out_type = (
        jax.ShapeDtypeStruct((B, N, 64, 128), jnp.float32),
        jax.ShapeDtypeStruct((B, N, 16, 2, 8, 128), jnp.float32),
    )
    mesh = plsc.VectorSubcoreMesh(core_axis_name="c", subcore_axis_name="s")
    sc_f = pl.kernel(
        _sc_body,
        out_type=sc_out_type,
        mesh=mesh,
        scratch_types=[
            pltpu.VMEM((C, N), jnp.float32),
            pltpu.VMEM((C // 8, 8, PW), jnp.float32),
            pltpu.VMEM((TW, C), jnp.float32),
            pltpu.SemaphoreType.DMA,
        ],
        compiler_params=pltpu.CompilerParams(
            use_tc_tiling_on_sc=False, needs_layout_passes=False),
    )
    o64t, o256t = sc_f(feature_seq)

    o128_sds = jax.ShapeDtypeStruct((B, N, 16, 8, 128), jnp.float32)
    o128a, o128b = pl.pallas_call(
        _tc_body,
        grid=(B, NT),
        in_specs=[pl.BlockSpec((1, C, N), lambda b, nt: (b, 0, 0))],
        out_specs=(
            pl.BlockSpec((1, CPB, 16, 8, 128), lambda b, nt: (b, nt, 0, 0, 0)),
            pl.BlockSpec((1, CPB, 16, 8, 128), lambda b, nt: (b, nt, 0, 0, 0)),
        ),
        out_shape=(o128_sds, o128_sds),
        scratch_shapes=[pltpu.VMEM((C, PW), jnp.float32)],
    )(feature_seq)
    return o64t, o256t, o128a, o128b


def kernel(feature_seq):
    o64t, o256t, o128a, o128b = _run(feature_seq)
    o64 = jnp.transpose(o64t, (0, 1, 3, 2))
    o256 = jnp.transpose(o256t, (0, 1, 2, 4, 3, 5)).reshape(B, N, C, 256)
    return (o64, o256,
            o128a.reshape(B, N, C, 128), o128b.reshape(B, N, C, 128))
